# baseline (speedup 1.0000x reference)
import functools

import jax
import jax.numpy as jnp
from jax import lax
from jax.experimental import pallas as pl
from jax.experimental.pallas import tpu as pltpu
from jax.experimental.pallas import tpu_sc as plsc

_NC = 2
_NS = 16
_NW = _NC * _NS

_CHUNK = 16
_NBUF = 2


def _sc_embedding_gather(weight, input_ids):
    batch, seq = input_ids.shape
    d_model = weight.shape[1]
    b_per_w = (batch * seq) // _NW
    n_chunks = b_per_w // _CHUNK
    n_groups = n_chunks // _NBUF
    w_per_row = seq // b_per_w

    mesh = plsc.VectorSubcoreMesh(core_axis_name="c", subcore_axis_name="s")

    @functools.partial(
        pl.kernel,
        mesh=mesh,
        out_type=jax.ShapeDtypeStruct((batch, seq, d_model), jnp.float32),
        scratch_types=(
            [pltpu.VMEM((b_per_w,), jnp.int32)]
            + [pltpu.VMEM((_CHUNK, d_model), jnp.float32)] * _NBUF
            + [pltpu.SemaphoreType.DMA] * (2 * _NBUF)
        ),
    )
    def k(table_hbm, idx_hbm, out_hbm, idx_v, *bufs_and_sems):
        bufs = list(bufs_and_sems[:_NBUF])
        gsems = list(bufs_and_sems[_NBUF:2 * _NBUF])
        wsems = list(bufs_and_sems[2 * _NBUF:])
        wid = lax.axis_index("s") * _NC + lax.axis_index("c")
        row = wid // w_per_row
        col0 = (wid % w_per_row) * b_per_w
        pltpu.sync_copy(idx_hbm.at[row, pl.ds(col0, b_per_w)], idx_v)

        def gather(c, b):
            off = pl.multiple_of(c * _CHUNK, 8)
            return pltpu.async_copy(
                table_hbm.at[idx_v.at[pl.ds(off, _CHUNK)]], bufs[b],
                gsems[b])

        def gather_wait(b):
            pltpu.make_async_copy(
                table_hbm.at[idx_v.at[pl.ds(0, _CHUNK)]], bufs[b],
                gsems[b]).wait()

        def write(c, b):
            off = pl.multiple_of(col0 + c * _CHUNK, 8)
            return pltpu.async_copy(
                bufs[b], out_hbm.at[row, pl.ds(off, _CHUNK)], wsems[b])

        for b in range(_NBUF):
            gather(b, b)

        @pl.loop(0, n_groups - 1, unroll=2)
        def _steady(g):
            for b in range(_NBUF):
                c = g * _NBUF + b
                gather_wait(b)
                write(c, b).wait()
                gather(c + _NBUF, b)

        for b in range(_NBUF):
            c = (n_groups - 1) * _NBUF + b
            gather_wait(b)
            write(c, b).wait()

    return k(weight, input_ids)


def kernel(input_ids, weight):
    return _sc_embedding_gather(weight, input_ids.astype(jnp.int32))


# speedup vs baseline: 1.0181x; 1.0181x over previous
import functools

import jax
import jax.numpy as jnp
from jax import lax
from jax.experimental import pallas as pl
from jax.experimental.pallas import tpu as pltpu
from jax.experimental.pallas import tpu_sc as plsc

_NC = 2
_NS = 16
_NW = _NC * _NS

_CHUNK = 16
_NBUF = 3


def _sc_embedding_gather(weight, input_ids):
    batch, seq = input_ids.shape
    d_model = weight.shape[1]
    b_per_w = (batch * seq) // _NW
    n_chunks = b_per_w // _CHUNK
    n_groups = n_chunks // _NBUF
    w_per_row = seq // b_per_w

    mesh = plsc.VectorSubcoreMesh(core_axis_name="c", subcore_axis_name="s")

    @functools.partial(
        pl.kernel,
        mesh=mesh,
        out_type=jax.ShapeDtypeStruct((batch, seq, d_model), jnp.float32),
        scratch_types=(
            [pltpu.VMEM((b_per_w,), jnp.int32)]
            + [pltpu.VMEM((_CHUNK, d_model), jnp.float32)] * _NBUF
            + [pltpu.SemaphoreType.DMA] * (2 * _NBUF)
        ),
    )
    def k(table_hbm, idx_hbm, out_hbm, idx_v, *bufs_and_sems):
        bufs = list(bufs_and_sems[:_NBUF])
        gsems = list(bufs_and_sems[_NBUF:2 * _NBUF])
        wsems = list(bufs_and_sems[2 * _NBUF:])
        wid = lax.axis_index("s") * _NC + lax.axis_index("c")
        row = wid // w_per_row
        col0 = (wid % w_per_row) * b_per_w
        pltpu.sync_copy(idx_hbm.at[row, pl.ds(col0, b_per_w)], idx_v)

        def gather(c, b):
            off = pl.multiple_of(c * _CHUNK, 8)
            return pltpu.async_copy(
                table_hbm.at[idx_v.at[pl.ds(off, _CHUNK)]], bufs[b],
                gsems[b])

        def gather_wait(b):
            pltpu.make_async_copy(
                table_hbm.at[idx_v.at[pl.ds(0, _CHUNK)]], bufs[b],
                gsems[b]).wait()

        def write(c, b):
            off = pl.multiple_of(col0 + c * _CHUNK, 8)
            return pltpu.async_copy(
                bufs[b], out_hbm.at[row, pl.ds(off, _CHUNK)], wsems[b])

        for b in range(_NBUF):
            gather(b, b)

        n_steady = (n_chunks - _NBUF) // _NBUF

        @pl.loop(0, n_steady)
        def _steady(g):
            for b in range(_NBUF):
                c = g * _NBUF + b
                gather_wait(b)
                write(c, b).wait()
                gather(c + _NBUF, b)

        for c in range(n_steady * _NBUF, n_chunks):
            b = c % _NBUF
            gather_wait(b)
            write(c, b).wait()
            if c + _NBUF < n_chunks:
                gather(c + _NBUF, b)

    return k(weight, input_ids)


def kernel(input_ids, weight):
    return _sc_embedding_gather(weight, input_ids.astype(jnp.int32))
